# Initial kernel scaffold; baseline (speedup 1.0000x reference)
import jax
import jax.numpy as jnp
from jax.experimental import pallas as pl


def kernel(tensor, biases, partitions, keys, W1, W2):
    raise NotImplementedError("write your pallas kernel here")


# speedup vs baseline: 1.8180x; 1.8180x over previous
import functools

import jax
import jax.numpy as jnp
from jax.experimental import pallas as pl
from jax.experimental.pallas import tpu as pltpu

B, T, D = 2, 2048, 1024
P = 8
K = 2
DFF = 1024
OFF_BIAS = 0.01
OFF_VAR = 0.01
NUDGE = 0.001

TB = 1024
N = B * T
NT = N // TB


def _fused_kernel(x_ref, keys_ref, bias_ref, w1_ref, w2_ref,
                  out_ref, psum_ref, cnt_ref, sq_ref, combine_ref):
    p = pl.program_id(1)

    @pl.when(p == 0)
    def _selector():
        x = x_ref[...]
        logits = jnp.dot(x, keys_ref[...].T,
                         preferred_element_type=jnp.float32) + bias_ref[...]
        m1 = jnp.max(logits, axis=1, keepdims=True)
        e = jnp.exp(logits - m1)
        probs = e / jnp.sum(e, axis=1, keepdims=True)
        iota = jax.lax.broadcasted_iota(jnp.int32, logits.shape, 1)
        arg1 = jnp.min(jnp.where(logits == m1, iota, P), axis=1, keepdims=True)
        masked = jnp.where(iota == arg1, -jnp.inf, logits)
        m2 = jnp.max(masked, axis=1, keepdims=True)
        arg2 = jnp.min(jnp.where(masked == m2, iota, P), axis=1, keepdims=True)
        w1v = 1.0 / (1.0 + jnp.exp(m2 - m1))
        sel1 = (iota == arg1).astype(jnp.float32)
        sel2 = (iota == arg2).astype(jnp.float32)
        combine_ref[...] = sel1 * w1v + sel2 * (1.0 - w1v)
        psum_ref[...] = jnp.sum(probs, axis=0, keepdims=True).reshape(1, 1, P)
        cnt_ref[...] = jnp.sum(sel1 + sel2, axis=0, keepdims=True).reshape(1, 1, P)
        sq_ref[...] = jnp.full((1, 1, P), jnp.sum(logits * logits), jnp.float32)

    x = x_ref[...]
    h = jnp.maximum(jnp.dot(x, w1_ref[0], preferred_element_type=jnp.float32), 0.0)
    y = jnp.dot(h, w2_ref[0], preferred_element_type=jnp.float32)
    iota = jax.lax.broadcasted_iota(jnp.int32, (TB, P), 1)
    c = jnp.sum(combine_ref[...] * (iota == p).astype(jnp.float32),
                axis=1, keepdims=True)
    y = y * c

    @pl.when(p == 0)
    def _init():
        out_ref[...] = y

    @pl.when(p > 0)
    def _acc():
        out_ref[...] += y


@jax.jit
def kernel(tensor, biases, partitions, keys, W1, W2):
    del partitions
    x = tensor.reshape(N, D)
    bias2d = biases.reshape(1, P)

    grid = (NT, P)
    out, psum, cnt, sq = pl.pallas_call(
        _fused_kernel,
        grid=grid,
        in_specs=[
            pl.BlockSpec((TB, D), lambda i, p: (i, 0)),
            pl.BlockSpec((P, D), lambda i, p: (0, 0)),
            pl.BlockSpec((1, P), lambda i, p: (0, 0)),
            pl.BlockSpec((1, D, DFF), lambda i, p: (p, 0, 0)),
            pl.BlockSpec((1, DFF, D), lambda i, p: (p, 0, 0)),
        ],
        out_specs=[
            pl.BlockSpec((TB, D), lambda i, p: (i, 0)),
            pl.BlockSpec((1, 1, P), lambda i, p: (i, 0, 0)),
            pl.BlockSpec((1, 1, P), lambda i, p: (i, 0, 0)),
            pl.BlockSpec((1, 1, P), lambda i, p: (i, 0, 0)),
        ],
        out_shape=[
            jax.ShapeDtypeStruct((N, D), jnp.float32),
            jax.ShapeDtypeStruct((NT, 1, P), jnp.float32),
            jax.ShapeDtypeStruct((NT, 1, P), jnp.float32),
            jax.ShapeDtypeStruct((NT, 1, P), jnp.float32),
        ],
        scratch_shapes=[pltpu.VMEM((TB, P), jnp.float32)],
    )(x, keys, bias2d, W1, W2)

    mean_prob = jnp.sum(psum, axis=(0, 1)) / N
    load_frac = jnp.sum(cnt, axis=(0, 1)) / (N * K)
    off_bias_loss = OFF_BIAS * P * jnp.sum(mean_prob * load_frac)
    off_var_loss = OFF_VAR * jnp.var(load_frac)
    nudge_loss = NUDGE * jnp.sum(sq[:, 0, 0]) / (N * P)
    loss = off_bias_loss + off_var_loss + nudge_loss
    return out.reshape(B, T, D), loss


# speedup vs baseline: 1.8198x; 1.0010x over previous
import functools

import jax
import jax.numpy as jnp
from jax.experimental import pallas as pl
from jax.experimental.pallas import tpu as pltpu

B, T, D = 2, 2048, 1024
P = 8
K = 2
DFF = 1024
OFF_BIAS = 0.01
OFF_VAR = 0.01
NUDGE = 0.001

TB = 1024
N = B * T
NT = N // TB


def _fused_kernel(x_ref, keys_ref, bias_ref, w1_ref, w2_ref,
                  out_ref, psum_ref, cnt_ref, sq_ref, combine_ref):
    p = pl.program_id(1)

    @pl.when(p == 0)
    def _selector():
        x = x_ref[...]
        logits = jnp.dot(x, keys_ref[...].T,
                         preferred_element_type=jnp.float32) + bias_ref[...]
        m1 = jnp.max(logits, axis=1, keepdims=True)
        e = jnp.exp(logits - m1)
        probs = e / jnp.sum(e, axis=1, keepdims=True)
        iota = jax.lax.broadcasted_iota(jnp.int32, logits.shape, 1)
        arg1 = jnp.min(jnp.where(logits == m1, iota, P), axis=1, keepdims=True)
        masked = jnp.where(iota == arg1, -jnp.inf, logits)
        m2 = jnp.max(masked, axis=1, keepdims=True)
        arg2 = jnp.min(jnp.where(masked == m2, iota, P), axis=1, keepdims=True)
        w1v = 1.0 / (1.0 + jnp.exp(m2 - m1))
        sel1 = (iota == arg1).astype(jnp.float32)
        sel2 = (iota == arg2).astype(jnp.float32)
        combine_ref[...] = sel1 * w1v + sel2 * (1.0 - w1v)
        psum_ref[...] = jnp.sum(probs, axis=0, keepdims=True).reshape(1, 1, P)
        cnt_ref[...] = jnp.sum(sel1 + sel2, axis=0, keepdims=True).reshape(1, 1, P)
        sq_ref[...] = jnp.full((1, 1, P), jnp.sum(logits * logits), jnp.float32)

    x = x_ref[...].astype(jnp.bfloat16)
    h = jnp.maximum(jnp.dot(x, w1_ref[0].astype(jnp.bfloat16),
                            preferred_element_type=jnp.float32), 0.0)
    y = jnp.dot(h.astype(jnp.bfloat16), w2_ref[0].astype(jnp.bfloat16),
                preferred_element_type=jnp.float32)
    iota = jax.lax.broadcasted_iota(jnp.int32, (TB, P), 1)
    c = jnp.sum(combine_ref[...] * (iota == p).astype(jnp.float32),
                axis=1, keepdims=True)
    y = y * c

    @pl.when(p == 0)
    def _init():
        out_ref[...] = y

    @pl.when(p > 0)
    def _acc():
        out_ref[...] += y


@jax.jit
def kernel(tensor, biases, partitions, keys, W1, W2):
    del partitions
    x = tensor.reshape(N, D)
    bias2d = biases.reshape(1, P)

    grid = (NT, P)
    out, psum, cnt, sq = pl.pallas_call(
        _fused_kernel,
        grid=grid,
        in_specs=[
            pl.BlockSpec((TB, D), lambda i, p: (i, 0)),
            pl.BlockSpec((P, D), lambda i, p: (0, 0)),
            pl.BlockSpec((1, P), lambda i, p: (0, 0)),
            pl.BlockSpec((1, D, DFF), lambda i, p: (p, 0, 0)),
            pl.BlockSpec((1, DFF, D), lambda i, p: (p, 0, 0)),
        ],
        out_specs=[
            pl.BlockSpec((TB, D), lambda i, p: (i, 0)),
            pl.BlockSpec((1, 1, P), lambda i, p: (i, 0, 0)),
            pl.BlockSpec((1, 1, P), lambda i, p: (i, 0, 0)),
            pl.BlockSpec((1, 1, P), lambda i, p: (i, 0, 0)),
        ],
        out_shape=[
            jax.ShapeDtypeStruct((N, D), jnp.float32),
            jax.ShapeDtypeStruct((NT, 1, P), jnp.float32),
            jax.ShapeDtypeStruct((NT, 1, P), jnp.float32),
            jax.ShapeDtypeStruct((NT, 1, P), jnp.float32),
        ],
        scratch_shapes=[pltpu.VMEM((TB, P), jnp.float32)],
    )(x, keys, bias2d, W1, W2)

    mean_prob = jnp.sum(psum, axis=(0, 1)) / N
    load_frac = jnp.sum(cnt, axis=(0, 1)) / (N * K)
    off_bias_loss = OFF_BIAS * P * jnp.sum(mean_prob * load_frac)
    off_var_loss = OFF_VAR * jnp.var(load_frac)
    nudge_loss = NUDGE * jnp.sum(sq[:, 0, 0]) / (N * P)
    loss = off_bias_loss + off_var_loss + nudge_loss
    return out.reshape(B, T, D), loss
